# baseline (speedup 1.0000x reference)
import functools

import jax
import jax.numpy as jnp
from jax import lax
from jax.experimental import pallas as pl
from jax.experimental.pallas import tpu as pltpu
from jax.experimental.pallas import tpu_sc as plsc

N = 10000
NP = 10240
D = 128
DH = 64
E = 320000
EP = 327680
CHUNK = 128
ECHUNKS = EP // CHUNK
ROWS_T = NP // 16
BLK = 512

_MESH = plsc.VectorSubcoreMesh(core_axis_name="c", subcore_axis_name="s",
                               num_cores=2, num_subcores=16)


@functools.partial(
    pl.kernel,
    out_type=jax.ShapeDtypeStruct((2, NP, 8), jnp.float32),
    mesh=_MESH,
    scratch_types=[
        pltpu.VMEM((ECHUNKS // 16, CHUNK), jnp.int32),
        pltpu.VMEM((CHUNK, 8), jnp.float32),
        pltpu.VMEM_SHARED((NP, 8), jnp.float32),
    ],
    compiler_params=pltpu.CompilerParams(use_tc_tiling_on_sc=False),
)
def _deg_kernel(idx2, ones_h, z8, out, idx_v, ones_v, acc):
    cid = lax.axis_index("c")
    sid = lax.axis_index("s")
    nck = ECHUNKS // 16
    pltpu.sync_copy(ones_h, ones_v)
    pltpu.sync_copy(z8, acc.at[pl.ds(sid * ROWS_T, ROWS_T)])
    pltpu.sync_copy(idx2.at[cid, pl.ds(sid * nck, nck)], idx_v)
    plsc.subcore_barrier()

    def body(c, carry):
        pltpu.sync_copy(ones_v, acc.at[idx_v.at[c]], add=True)
        return carry

    lax.fori_loop(0, nck, body, 0)
    plsc.subcore_barrier()
    pltpu.sync_copy(acc.at[pl.ds(sid * ROWS_T, ROWS_T)],
                    out.at[cid, pl.ds(sid * ROWS_T, ROWS_T)])


NBUF = 6
GDIST = 4


@functools.partial(
    pl.kernel,
    out_type=jax.ShapeDtypeStruct((2, NP, DH), jnp.float32),
    mesh=_MESH,
    scratch_types=[
        pltpu.VMEM((ECHUNKS // 16, CHUNK), jnp.int32),
        pltpu.VMEM((ECHUNKS // 16, CHUNK), jnp.int32),
        pltpu.VMEM((NBUF, CHUNK, DH), jnp.float32),
        pltpu.VMEM_SHARED((NP, DH), jnp.float32),
    ] + [pltpu.SemaphoreType.DMA] * (2 * NBUF),
    compiler_params=pltpu.CompilerParams(use_tc_tiling_on_sc=False),
)
def _agg_kernel(hh, srci, dstp, zrows, out,
                si_v, di_v, buf_v, acc_sh, *sems):
    cid = lax.axis_index("c")
    sid = lax.axis_index("s")
    r0 = sid * ROWS_T
    nck = ECHUNKS // 16
    pltpu.sync_copy(zrows, acc_sh.at[pl.ds(r0, ROWS_T)])
    pltpu.sync_copy(srci.at[cid, pl.ds(sid * nck, nck)], si_v)
    pltpu.sync_copy(dstp.at[pl.ds(sid * nck, nck)], di_v)
    plsc.subcore_barrier()

    sg = sems[:NBUF]
    ss = sems[NBUF:]

    def start_g(c, b):
        pltpu.async_copy(hh.at[si_v.at[c]], buf_v.at[b], sg[b])

    def wait_g(b):
        pltpu.make_async_copy(hh.at[si_v.at[0]], buf_v.at[b], sg[b]).wait()

    def start_s(c, b):
        pltpu.async_copy(buf_v.at[b], acc_sh.at[di_v.at[c]], ss[b], add=True)

    def wait_s(b):
        pltpu.make_async_copy(buf_v.at[b], acc_sh.at[di_v.at[0]],
                              ss[b]).wait()

    nfull = nck - 2 - GDIST
    main_iters = nfull // NBUF
    for c in range(GDIST):
        start_g(c, c)
    for c in range(2):
        wait_g(c)
        start_s(c, c)
        start_g(c + GDIST, (c + GDIST) % NBUF)

    def loop_body(k, carry):
        for j in range(NBUF):
            c = NBUF * k + 2 + j
            b = (2 + j) % NBUF
            wait_g(b)
            start_s(c, b)
            wait_s((b + GDIST) % NBUF)
            start_g(c + GDIST, (b + GDIST) % NBUF)
        return carry

    lax.fori_loop(0, main_iters, loop_body, 0)
    for c in range(2 + main_iters * NBUF, nck - GDIST):
        b = c % NBUF
        wait_g(b)
        start_s(c, b)
        wait_s((b + GDIST) % NBUF)
        start_g(c + GDIST, (b + GDIST) % NBUF)
    for c in range(nck - GDIST, nck):
        b = c % NBUF
        wait_g(b)
        start_s(c, b)
        wait_s((b + GDIST) % NBUF)
    wait_s((nck - 2) % NBUF)
    wait_s((nck - 1) % NBUF)
    plsc.subcore_barrier()
    pltpu.sync_copy(acc_sh.at[pl.ds(r0, ROWS_T)],
                    out.at[cid, pl.ds(r0, ROWS_T)])


def _prep_body(x_ref, ds_ref, dd_ref, hh_ref, ns_ref, nd_ref):
    deg_s = ds_ref[...][:, 0:1]
    deg_d = dd_ref[...][:, 0:1]
    ns = jnp.where(deg_s > 0, lax.rsqrt(deg_s), 0.0)
    nd = jnp.where(deg_d > 0, lax.rsqrt(deg_d), 0.0)
    h = x_ref[...] * ns
    hh_ref[0] = h[:, :DH]
    hh_ref[1] = h[:, DH:]
    ns_ref[...] = ns
    nd_ref[...] = nd


def _prep_stage(xp, deg_s, deg_d):
    return pl.pallas_call(
        _prep_body,
        grid=(NP // BLK,),
        in_specs=[
            pl.BlockSpec((BLK, D), lambda i: (i, 0)),
            pl.BlockSpec((BLK, 8), lambda i: (i, 0)),
            pl.BlockSpec((BLK, 8), lambda i: (i, 0)),
        ],
        out_specs=[
            pl.BlockSpec((2, BLK, DH), lambda i: (0, i, 0)),
            pl.BlockSpec((BLK, 1), lambda i: (i, 0)),
            pl.BlockSpec((BLK, 1), lambda i: (i, 0)),
        ],
        out_shape=[
            jax.ShapeDtypeStruct((2, NP, DH), jnp.float32),
            jax.ShapeDtypeStruct((NP, 1), jnp.float32),
            jax.ShapeDtypeStruct((NP, 1), jnp.float32),
        ],
    )(xp, deg_s, deg_d)


def _dense_body(p_ref, nd_ref, w_ref, b_ref, *rest, relu, post, split_out):
    if post:
        ns_ref = rest[0]
        rest = rest[1:]
    o_ref = rest[0]
    pv = p_ref[...]
    agg = jnp.concatenate([pv[0], pv[1]], axis=1)
    y = jnp.dot(agg * nd_ref[...], w_ref[...],
                preferred_element_type=jnp.float32) + b_ref[...]
    if relu:
        y = jnp.maximum(y, 0.0)
    if post:
        y = y * ns_ref[...]
    if split_out:
        o_ref[0] = y[:, :DH]
        o_ref[1] = y[:, DH:]
    else:
        o_ref[...] = y


def _dense_stage(p, nd, w, b, relu, post=None, split_out=False):
    args = [p, nd, w, b]
    in_specs = [
        pl.BlockSpec((2, BLK, DH), lambda i: (0, i, 0)),
        pl.BlockSpec((BLK, 1), lambda i: (i, 0)),
        pl.BlockSpec((D, D), lambda i: (0, 0)),
        pl.BlockSpec((1, D), lambda i: (0, 0)),
    ]
    if post is not None:
        args.append(post)
        in_specs.append(pl.BlockSpec((BLK, 1), lambda i: (i, 0)))
    if split_out:
        out_spec = pl.BlockSpec((2, BLK, DH), lambda i: (0, i, 0))
        out_shape = jax.ShapeDtypeStruct((2, NP, DH), jnp.float32)
    else:
        out_spec = pl.BlockSpec((BLK, D), lambda i: (i, 0))
        out_shape = jax.ShapeDtypeStruct((NP, D), jnp.float32)
    body = functools.partial(_dense_body, relu=relu, post=post is not None,
                             split_out=split_out)
    return pl.pallas_call(
        body,
        grid=(NP // BLK,),
        in_specs=in_specs,
        out_specs=out_spec,
        out_shape=out_shape,
    )(*args)


def kernel(x, edge_index, W1, b1, W2, b2):
    prev_x64 = jax.config.jax_enable_x64
    jax.config.update("jax_enable_x64", False)
    try:
        return _kernel_impl(x, edge_index, W1, b1, W2, b2)
    finally:
        jax.config.update("jax_enable_x64", prev_x64)


def _kernel_impl(x, edge_index, W1, b1, W2, b2):
    src = edge_index[0].astype(jnp.int32)
    dst = edge_index[1].astype(jnp.int32)
    x = x.astype(jnp.float32)
    W1 = W1.astype(jnp.float32)
    W2 = W2.astype(jnp.float32)
    b1 = b1.astype(jnp.float32).reshape(1, D)
    b2 = b2.astype(jnp.float32).reshape(1, D)

    pad = jnp.full((EP - E,), N, jnp.int32)
    srcp = jnp.concatenate([src, pad]).reshape(ECHUNKS, CHUNK)
    dstp = jnp.concatenate([dst, pad]).reshape(ECHUNKS, CHUNK)
    xp = jnp.zeros((NP, D), jnp.float32).at[:N].set(x)

    idx2 = jnp.stack([srcp, dstp])
    src2 = jnp.stack([srcp, srcp + NP])
    ones_h = jnp.ones((CHUNK, 8), jnp.float32)
    z8 = jnp.zeros((ROWS_T, 8), jnp.float32)
    zrows = jnp.zeros((ROWS_T, DH), jnp.float32)

    deg = _deg_kernel(idx2, ones_h, z8)
    hh, ns, nd = _prep_stage(xp, deg[0], deg[1])
    p1 = _agg_kernel(hh.reshape(2 * NP, DH), src2, dstp, zrows)
    h1h = _dense_stage(p1, nd, W1, b1, relu=True, post=ns, split_out=True)
    p2 = _agg_kernel(h1h.reshape(2 * NP, DH), src2, dstp, zrows)
    out = _dense_stage(p2, nd, W2, b2, relu=False)
    return out[:N]
